# Initial kernel scaffold; baseline (speedup 1.0000x reference)
import jax
import jax.numpy as jnp
from jax.experimental import pallas as pl


def kernel(feature_matrices, adj_matrices, adj_weights, d_model, W1, b1, W2, b2, ca_w1, ca_b1, ca_w2, ca_b2, cnn_w, cnn_b):
    raise NotImplementedError("write your pallas kernel here")


# speedup vs baseline: 7.9562x; 7.9562x over previous
import functools

import jax
import jax.numpy as jnp
from jax import lax
from jax.experimental import pallas as pl
from jax.experimental.pallas import tpu as pltpu
from jax.experimental.pallas import tpu_sc as plsc

M = 3
N = 10000
F = 128
E = 320000
NP = 10240
NC = 2
NS = 16
NW = NC * NS
K = 128
NCH = 82
HCH = NCH // 2
EPT = NCH * K
EPAD = NW * EPT
RPT = NP // NS
DPT = M * NP // NS
BN_EPS = 1e-5

_mesh = plsc.VectorSubcoreMesh(core_axis_name="c", subcore_axis_name="s")


@functools.partial(
    pl.kernel,
    out_type=jax.ShapeDtypeStruct((NC, M * NP), jnp.float32),
    mesh=_mesh,
    scratch_types=[
        pltpu.VMEM((HCH, K), jnp.int32),
        pltpu.VMEM((HCH, K), jnp.float32),
        pltpu.VMEM((DPT,), jnp.float32),
        pltpu.VMEM_SHARED((M * NP,), jnp.float32),
    ],
)
def _sc_degree(dst_h, w_h, out_h, dst_v, w_v, zb, acc):
    cid = lax.axis_index("c")
    sid = lax.axis_index("s")
    wid = cid * NS + sid

    def _zero(i, _):
        zb[pl.ds(i * 16, 16)] = jnp.zeros((16,), jnp.float32)
        return 0

    lax.fori_loop(0, DPT // 16, _zero, 0)
    pltpu.sync_copy(zb, acc.at[pl.ds(sid * DPT, DPT)])
    plsc.subcore_barrier()

    for m in range(M):
        for half in range(2):
            pltpu.sync_copy(dst_h.at[wid, m, half], dst_v)
            pltpu.sync_copy(w_h.at[wid, m, half], w_v)

            def _scat(j, _):
                pltpu.sync_copy(w_v.at[j], acc.at[dst_v.at[j]], add=True)
                return 0

            lax.fori_loop(0, HCH, _scat, 0)
    plsc.subcore_barrier()
    pltpu.sync_copy(acc.at[pl.ds(sid * DPT, DPT)], zb)
    pltpu.sync_copy(zb, out_h.at[cid, pl.ds(sid * DPT, DPT)])


@functools.partial(
    pl.kernel,
    out_type=jax.ShapeDtypeStruct((NC, M, NP, F), jnp.float32),
    mesh=_mesh,
    scratch_types=[
        pltpu.VMEM((HCH, K), jnp.int32),
        pltpu.VMEM((HCH, K), jnp.int32),
        pltpu.VMEM((HCH, K), jnp.float32),
        pltpu.VMEM((K, F), jnp.float32),
        pltpu.VMEM_SHARED((NP, F), jnp.float32),
        pltpu.SemaphoreType.DMA,
    ],
)
def _sc_edge_accum(xs_h, src_h, dst_h, w_h, out_h,
                   src_v, dst_v, w_v, rows, acc, sem):
    cid = lax.axis_index("c")
    sid = lax.axis_index("s")
    wid = cid * NS + sid

    for m in range(M):
        def _zero(r, _):
            for f in range(F // 16):
                rows[r, pl.ds(f * 16, 16)] = jnp.zeros((16,), jnp.float32)
            return 0

        lax.fori_loop(0, K, _zero, 0)
        for h in range(RPT // K):
            pltpu.sync_copy(rows, acc.at[pl.ds(sid * RPT + h * K, K)])
        plsc.subcore_barrier()

        for half in range(2):
            pltpu.sync_copy(src_h.at[wid, m, half], src_v)
            pltpu.sync_copy(dst_h.at[wid, m, half], dst_v)
            pltpu.sync_copy(w_h.at[wid, m, half], w_v)

            def _chunk(j, _):
                pltpu.async_copy(xs_h.at[src_v.at[j]], rows, sem).wait()

                def _scale(k16, _2):
                    wn = w_v[j, pl.ds(k16 * 16, 16)]
                    for e in range(16):
                        wk = wn[e]
                        r = k16 * 16 + e
                        for f in range(F // 16):
                            rows[r, pl.ds(f * 16, 16)] = (
                                rows[r, pl.ds(f * 16, 16)] * wk)
                    return 0

                lax.fori_loop(0, K // 16, _scale, 0)
                pltpu.sync_copy(rows, acc.at[dst_v.at[j]], add=True)
                return 0

            lax.fori_loop(0, HCH, _chunk, 0)
        plsc.subcore_barrier()

        for h in range(RPT // K):
            pltpu.sync_copy(acc.at[pl.ds(sid * RPT + h * K, K)], rows)
            pltpu.sync_copy(rows, out_h.at[cid, m, pl.ds(sid * RPT + h * K, K)])


def _tc_prep_body(deg_ref, x_ref, dinv_ref, xs1_ref):
    d = deg_ref[0, 0:1, :] + deg_ref[0, 1:2, :]
    dinv = jnp.where(d > 0, lax.rsqrt(d), 0.0)
    dinv_ref[0] = dinv
    dit = jnp.transpose(dinv[:, :N], (1, 0))
    xs1_ref[0] = dit * x_ref[0]


def _tc_mid_body(acc_ref, dinv_ref, w1_ref, w2_ref, xs2_ref):
    dit = jnp.transpose(dinv_ref[0, :, :N], (1, 0))
    agg = dit * (acc_ref[0, 0] + acc_ref[1, 0])
    y = jnp.dot(agg, w1_ref[...], preferred_element_type=jnp.float32)
    mu = jnp.mean(y, axis=0, keepdims=True)
    var = jnp.mean((y - mu) ** 2, axis=0, keepdims=True)
    x1 = jnp.maximum((y - mu) * lax.rsqrt(var + BN_EPS), 0.0)
    h2 = jnp.dot(x1, w2_ref[...], preferred_element_type=jnp.float32)
    xs2_ref[0] = dit * h2


def _tc_bn2_body(acc_ref, dinv_ref, x2_ref):
    dit = jnp.transpose(dinv_ref[0, :, :N], (1, 0))
    a = dit * (acc_ref[0, 0] + acc_ref[1, 0])
    mu = jnp.mean(a, axis=0, keepdims=True)
    var = jnp.mean((a - mu) ** 2, axis=0, keepdims=True)
    x2_ref[0] = jnp.maximum((a - mu) * lax.rsqrt(var + BN_EPS), 0.0)


def _tc_head_body(x2_ref, ca_w1_ref, ca_b1_ref, ca_w2_ref, ca_b2_ref,
                  wt_ref, cnn_b_ref, out_ref):
    pooled = [jnp.mean(x2_ref[m]) for m in range(M)]
    y1 = ca_b1_ref[...]
    for m in range(M):
        y1 = y1 + pooled[m] * ca_w1_ref[m:m + 1, :]
    y1 = jnp.maximum(y1, 0.0)
    z = jnp.sum(y1.T * ca_w2_ref[...], axis=0, keepdims=True) + ca_b2_ref[...]
    y = jax.nn.sigmoid(z)
    wty = y.T[:, :, None] * wt_ref[...]
    res = cnn_b_ref[...]
    for m in range(M):
        res = res + jnp.dot(x2_ref[m], wty[m],
                            preferred_element_type=jnp.float32)
    out_ref[...] = res


def kernel(feature_matrices, adj_matrices, adj_weights, d_model, W1, b1, W2,
           b2, ca_w1, ca_b1, ca_w2, ca_b2, cnn_w, cnn_b):
    del d_model, b1, b2
    f32 = jnp.float32

    src = adj_matrices[:, 0, :].astype(jnp.int32)
    dst = adj_matrices[:, 1, :].astype(jnp.int32)
    w = adj_weights.astype(f32)

    loop = jnp.broadcast_to(jnp.arange(N, dtype=jnp.int32), (M, N))
    padi = jnp.zeros((M, EPAD - E - N), jnp.int32)
    padf = jnp.zeros((M, EPAD - E - N), f32)
    src_p = jnp.concatenate([src, loop, padi], axis=1)
    dst_p = jnp.concatenate([dst, loop, padi], axis=1)
    w_p = jnp.concatenate([w, jnp.ones((M, N), f32), padf], axis=1)

    moff_n = (jnp.arange(M, dtype=jnp.int32) * N)[:, None]
    moff_p = (jnp.arange(M, dtype=jnp.int32) * NP)[:, None]

    def edge_layout(a):
        return a.reshape(M, NW, 2, HCH, K).transpose(1, 0, 2, 3, 4)

    src_off = edge_layout(src_p + moff_n)
    dst_plain = edge_layout(dst_p)
    dst_off = edge_layout(dst_p + moff_p)
    w_r = edge_layout(w_p)

    deg_parts = _sc_degree(dst_off, w_r).reshape(NC, M, NP).transpose(1, 0, 2)
    dinv, xs1 = pl.pallas_call(
        _tc_prep_body,
        grid=(M,),
        in_specs=[
            pl.BlockSpec((1, NC, NP), lambda m: (m, 0, 0)),
            pl.BlockSpec((1, N, F), lambda m: (m, 0, 0)),
        ],
        out_specs=[
            pl.BlockSpec((1, 1, NP), lambda m: (m, 0, 0)),
            pl.BlockSpec((1, N, F), lambda m: (m, 0, 0)),
        ],
        out_shape=[
            jax.ShapeDtypeStruct((M, 1, NP), f32),
            jax.ShapeDtypeStruct((M, N, F), f32),
        ],
    )(deg_parts, feature_matrices)

    acc1 = _sc_edge_accum(xs1.reshape(M * N, F),
                          src_off, dst_plain, w_r)[:, :, :N, :]

    xs2 = pl.pallas_call(
        _tc_mid_body,
        grid=(M,),
        in_specs=[
            pl.BlockSpec((NC, 1, N, F), lambda m: (0, m, 0, 0)),
            pl.BlockSpec((1, 1, NP), lambda m: (m, 0, 0)),
            pl.BlockSpec((F, 2 * F), lambda m: (0, 0)),
            pl.BlockSpec((2 * F, F), lambda m: (0, 0)),
        ],
        out_specs=pl.BlockSpec((1, N, F), lambda m: (m, 0, 0)),
        out_shape=jax.ShapeDtypeStruct((M, N, F), f32),
    )(acc1, dinv, W1, W2)

    acc2 = _sc_edge_accum(xs2.reshape(M * N, F),
                          src_off, dst_plain, w_r)[:, :, :N, :]

    x2 = pl.pallas_call(
        _tc_bn2_body,
        grid=(M,),
        in_specs=[
            pl.BlockSpec((NC, 1, N, F), lambda m: (0, m, 0, 0)),
            pl.BlockSpec((1, 1, NP), lambda m: (m, 0, 0)),
        ],
        out_specs=pl.BlockSpec((1, N, F), lambda m: (m, 0, 0)),
        out_shape=jax.ShapeDtypeStruct((M, N, F), f32),
    )(acc2, dinv)

    wt = jnp.transpose(cnn_w[..., 0], (1, 2, 0))
    result = pl.pallas_call(
        _tc_head_body,
        out_shape=jax.ShapeDtypeStruct((N, F), f32),
    )(x2, ca_w1, ca_b1.reshape(1, -1), ca_w2, ca_b2.reshape(1, -1), wt,
      cnn_b.reshape(1, -1))
    return result
